# Initial kernel scaffold; baseline (speedup 1.0000x reference)
import jax
import jax.numpy as jnp
from jax.experimental import pallas as pl


def kernel(idx, table, W, b):
    raise NotImplementedError("write your pallas kernel here")


# speedup vs baseline: 1.0722x; 1.0722x over previous
import functools

import jax
import jax.numpy as jnp
from jax import lax
from jax.experimental import pallas as pl
from jax.experimental.pallas import tpu as pltpu
from jax.experimental.pallas import tpu_sc as plsc

_PHI = 1.61803398875
_D = 64
_CHUNK = 512


@functools.lru_cache(maxsize=None)
def _make_gather(B, Vhalf):
    info = plsc.get_sparse_core_info()
    nc, ns = info.num_cores, info.num_subcores
    nw = nc * ns
    b_per_w = B // nw
    n_chunks = b_per_w // _CHUNK
    mesh = plsc.VectorSubcoreMesh(core_axis_name="c", subcore_axis_name="s")

    @functools.partial(
        pl.kernel,
        mesh=mesh,
        out_type=jax.ShapeDtypeStruct((B, 2 * _D), jnp.float32),
        scratch_types=[
            pltpu.VMEM((b_per_w,), jnp.int32),
            pltpu.VMEM((_CHUNK, 2 * _D), jnp.float32),
            pltpu.SemaphoreType.DMA,
        ],
    )
    def gather(idx_hbm, table_hbm, out_hbm, idx_v, rows_v, sem):
        wid = lax.axis_index("s") * nc + lax.axis_index("c")
        base = wid * b_per_w
        pltpu.sync_copy(idx_hbm.at[pl.ds(base, b_per_w)], idx_v)

        def body(c, carry):
            off = c * _CHUNK
            pltpu.async_copy(
                table_hbm.at[idx_v.at[pl.ds(off, _CHUNK)]], rows_v, sem
            ).wait()
            pltpu.sync_copy(rows_v, out_hbm.at[pl.ds(base + off, _CHUNK)])
            return carry

        lax.fori_loop(0, n_chunks, body, 0)

    return gather


def _mm_body(g_ref, p_ref, w_ref, b_ref, o_ref):
    lo = g_ref[:, :_D]
    hi = g_ref[:, _D:]
    x = lo + (hi - lo) * p_ref[...]
    acc = lax.dot_general(
        x, w_ref[...], (((1,), (1,)), ((), ())),
        preferred_element_type=jnp.float32,
    )
    o_ref[...] = (acc + b_ref[...]) * _PHI


def _transform(g, parity, W, b):
    B = g.shape[0]
    block = 4096
    return pl.pallas_call(
        _mm_body,
        grid=(B // block,),
        in_specs=[
            pl.BlockSpec((block, 2 * _D), lambda i: (i, 0)),
            pl.BlockSpec((block, 1), lambda i: (i, 0)),
            pl.BlockSpec((_D, _D), lambda i: (0, 0)),
            pl.BlockSpec((1, _D), lambda i: (0, 0)),
        ],
        out_specs=pl.BlockSpec((block, _D), lambda i: (i, 0)),
        out_shape=jax.ShapeDtypeStruct((B, _D), jnp.float32),
    )(g, parity, W, b)


def kernel(idx, table, W, b):
    Bo, L = idx.shape
    flat_idx = idx.reshape(-1).astype(jnp.int32)
    B = flat_idx.shape[0]
    half_idx = lax.shift_right_logical(flat_idx, 1)
    parity = (flat_idx & 1).astype(jnp.float32).reshape(B, 1)
    table2 = table.reshape(table.shape[0] // 2, 2 * _D)
    gathered = _make_gather(B, table2.shape[0])(half_idx, table2)
    out = _transform(gathered, parity, W, b.reshape(1, _D))
    return out.reshape(Bo, L, _D)
